# baseline (speedup 1.0000x reference)
import functools

import jax
import jax.numpy as jnp
from jax import lax
from jax.experimental import pallas as pl
from jax.experimental.pallas import tpu as pltpu
from jax.experimental.pallas import tpu_sc as plsc

NUM_ENT = 1000000
NUM_REL = 1000
DIM = 64
B = 16384

NC = 2
NS = 16
NW = NC * NS
BW = B // NW
CH = 64
NCH = BW // CH

TCOLS = 16384
RSIDE = 23 * TCOLS
TOFF = RSIDE // TCOLS
TSIDE = NUM_ENT - RSIDE
SLAB = 256
NCHK = RSIDE // SLAB
CPT = NCHK // NW

LN2 = 0.6931471805599453
SQRT2 = 1.4142135623730951


def _ln(x):
    xb = lax.bitcast_convert_type(x, jnp.int32)
    e = lax.shift_right_arithmetic(xb, 23) - 127
    mb = lax.bitwise_or(lax.bitwise_and(xb, 0x007FFFFF), 0x3F800000)
    m = lax.bitcast_convert_type(mb, jnp.float32)
    big = m > SQRT2
    m = jnp.where(big, m * 0.5, m)
    ef = e.astype(jnp.float32) + jnp.where(big, 1.0, 0.0)
    s = (m - 1.0) / (m + 1.0)
    u = s * s
    p = 2.0 + u * (0.6666666666 + u * (0.4000000897 + u * (0.2857142857 + u * 0.2222222222)))
    return ef * LN2 + s * p


def _energy_group(hmu, tmu, rmu, hsd, tsd, rsd):
    h_sig = jnp.abs(hsd) + 1e-6
    t_sig = jnp.abs(tsd) + 1e-6
    r_sig = jnp.abs(rsd) + 1e-6
    pred_sig = h_sig + r_sig + 1e-6
    inv_t = 1.0 / t_sig
    ratio = pred_sig * inv_t
    d = tmu - (hmu + rmu)
    return ratio + d * d * inv_t - _ln(ratio)


def _cat_body(a_ref, b_ref, o_ref, x_ref):
    x_ref[:DIM] = a_ref[...]
    x_ref[DIM:] = b_ref[...]
    o_ref[...] = x_ref[...].T


def _make_cat(n, cols, off=0):
    return pl.pallas_call(
        _cat_body,
        grid=(pl.cdiv(n, cols),),
        in_specs=[
            pl.BlockSpec((DIM, cols), lambda i: (0, i + off)),
            pl.BlockSpec((DIM, cols), lambda i: (0, i + off)),
        ],
        out_specs=pl.BlockSpec((cols, 2 * DIM), lambda i: (i, 0)),
        out_shape=jax.ShapeDtypeStruct((n, 2 * DIM), jnp.float32),
        scratch_shapes=[pltpu.VMEM((2 * DIM, cols), jnp.float32)],
    )


def _relay_body(edT_hbm, esT_hbm, outB_hbm, slabA, slabB, orow):
    wid = lax.axis_index("c") * NS + lax.axis_index("s")
    lanes = lax.iota(jnp.int32, 16)
    chunk0 = wid * CPT

    def do_slab(row0):
        col0 = pl.multiple_of(row0, 128)
        pltpu.sync_copy(edT_hbm.at[:, pl.ds(col0, SLAB)], slabA)
        pltpu.sync_copy(esT_hbm.at[:, pl.ds(col0, SLAB)], slabB)

        def row_body(e, _):
            ev = jnp.full((16,), e, jnp.int32)
            for g in range(DIM // 16):
                rows = g * 16 + lanes
                orow[e, pl.ds(g * 16, 16)] = plsc.load_gather(slabA, [rows, ev])
                orow[e, pl.ds(DIM + g * 16, 16)] = plsc.load_gather(slabB, [rows, ev])
            return 0

        lax.fori_loop(0, SLAB, row_body, 0)
        pltpu.sync_copy(orow, outB_hbm.at[pl.ds(row0, SLAB)])

    for k in range(CPT):
        do_slab((chunk0 + k) * SLAB)


def _issue(entA, entB, rel, ha, hb, ta, tb, r, c, bufs, sem):
    bha, bhb, bta, btb, br = bufs
    return [
        pltpu.async_copy(entA.at[ha.at[c]], bha, sem),
        pltpu.async_copy(entB.at[hb.at[c]], bhb, sem),
        pltpu.async_copy(entA.at[ta.at[c]], bta, sem),
        pltpu.async_copy(entB.at[tb.at[c]], btb, sem),
        pltpu.async_copy(rel.at[r.at[c]], br, sem),
    ]


def _sc_body(ha_hbm, hb_hbm, ta_hbm, tb_hbm, r_hbm, pk_hbm,
             entA_hbm, entB_hbm, rel_hbm, out_hbm,
             ha, hb, ta, tb, ri, pkv, b0, b1, accs_f, out_v, sem0, sem1):
    wid = lax.axis_index("c") * NS + lax.axis_index("s")
    lanes = lax.iota(jnp.int32, 16)

    pltpu.sync_copy(ha_hbm.at[wid], ha)
    pltpu.sync_copy(hb_hbm.at[wid], hb)
    pltpu.sync_copy(ta_hbm.at[wid], ta)
    pltpu.sync_copy(tb_hbm.at[wid], tb)
    pltpu.sync_copy(r_hbm.at[wid], ri)
    pltpu.sync_copy(pk_hbm.at[wid], pkv.at[pl.ds(0, BW)])

    slots = ((b0, sem0), (b1, sem1))
    pending = _issue(entA_hbm, entB_hbm, rel_hbm, ha, hb, ta, tb, ri,
                     0, b0, sem0)

    for c in range(NCH):
        bufs, _ = slots[c % 2]
        bha, bhb, bta, btb, br = bufs
        for cp in pending:
            cp.wait()
        if c + 1 < NCH:
            nbufs, nsem = slots[(c + 1) % 2]
            pending = _issue(entA_hbm, entB_hbm, rel_hbm, ha, hb, ta, tb, ri,
                             c + 1, nbufs, nsem)

        def row_body(b, _):
            pk = pkv[pl.ds(c * CH + b, 16)][0]
            mh = jnp.full((16,), pk & 1, jnp.int32) != 0
            mt = jnp.full((16,), pk & 2, jnp.int32) != 0
            acc = jnp.zeros((16,), jnp.float32)
            for g in range(DIM // 16):
                mu = pl.ds(g * 16, 16)
                sd = pl.ds(DIM + g * 16, 16)
                hmu = jnp.where(mh, bhb[b, mu], bha[b, mu])
                hsd = jnp.where(mh, bhb[b, sd], bha[b, sd])
                tmu = jnp.where(mt, btb[b, mu], bta[b, mu])
                tsd = jnp.where(mt, btb[b, sd], bta[b, sd])
                acc = acc + _energy_group(hmu, tmu, br[b, mu],
                                          hsd, tsd, br[b, sd])
            accs_f[pl.ds(b * 16, 16)] = acc
            return 0

        lax.fori_loop(0, CH, row_body, 0)

        for bb in range(CH // 16):
            tot = jnp.zeros((16,), jnp.float32)
            for j in range(16):
                col = plsc.load_gather(accs_f, [bb * 256 + lanes * 16 + j])
                tot = tot + col
            out_v[pl.ds(c * CH + bb * 16, 16)] = 0.5 * tot

    pltpu.sync_copy(out_v, out_hbm.at[pl.ds(wid * BW, BW)])


@jax.jit
def _run(ha2, hb2, ta2, tb2, r2, pk2, ent_drift, ent_diff, rel_drift, rel_diff):
    mesh = plsc.VectorSubcoreMesh(core_axis_name="c", subcore_axis_name="s")
    scparams = pltpu.CompilerParams(needs_layout_passes=False)

    edT, esT = ent_drift.T, ent_diff.T
    entA = _make_cat(TSIDE, TCOLS, TOFF)(edT, esT)
    rel_cat = _make_cat(NUM_REL, NUM_REL)(rel_drift.T, rel_diff.T)

    relay = functools.partial(
        pl.kernel,
        out_type=jax.ShapeDtypeStruct((RSIDE, 2 * DIM), jnp.float32),
        mesh=mesh,
        compiler_params=scparams,
        scratch_types=[
            pltpu.VMEM((DIM, SLAB), jnp.float32),
            pltpu.VMEM((DIM, SLAB), jnp.float32),
            pltpu.VMEM((SLAB, 2 * DIM), jnp.float32),
        ],
    )(_relay_body)
    entB = relay(edT, esT)

    buf = lambda: pltpu.VMEM((CH, 2 * DIM), jnp.float32)
    idx = lambda: pltpu.VMEM((NCH, CH), jnp.int32)
    kfn = functools.partial(
        pl.kernel,
        out_type=jax.ShapeDtypeStruct((B,), jnp.float32),
        mesh=mesh,
        compiler_params=scparams,
        scratch_types=[
            idx(), idx(), idx(), idx(), idx(),
            pltpu.VMEM((BW + 16,), jnp.int32),
            (buf(), buf(), buf(), buf(), buf()),
            (buf(), buf(), buf(), buf(), buf()),
            pltpu.VMEM((CH * 16,), jnp.float32),
            pltpu.VMEM((BW,), jnp.float32),
            pltpu.SemaphoreType.DMA,
            pltpu.SemaphoreType.DMA,
        ],
    )(_sc_body)
    return kfn(ha2, hb2, ta2, tb2, r2, pk2, entA, entB, rel_cat)


def kernel(h_idx, r_idx, t_idx, ent_drift, ent_diff, rel_drift, rel_diff):
    h_idx = h_idx.astype(jnp.int32)
    t_idx = t_idx.astype(jnp.int32)
    r_idx = r_idx.astype(jnp.int32)
    hB = h_idx < RSIDE
    tB = t_idx < RSIDE
    sh = (NW, NCH, CH)
    ha2 = jnp.where(hB, 0, h_idx - RSIDE).reshape(sh)
    hb2 = jnp.where(hB, h_idx, 0).reshape(sh)
    ta2 = jnp.where(tB, 0, t_idx - RSIDE).reshape(sh)
    tb2 = jnp.where(tB, t_idx, 0).reshape(sh)
    r2 = r_idx.reshape(sh)
    pk2 = (hB.astype(jnp.int32) | (tB.astype(jnp.int32) << 1)).reshape(NW, BW)
    return _run(ha2, hb2, ta2, tb2, r2, pk2,
                ent_drift, ent_diff, rel_drift, rel_diff)


# speedup vs baseline: 5.8494x; 5.8494x over previous
import functools

import jax
import jax.numpy as jnp
from jax import lax
from jax.experimental import pallas as pl
from jax.experimental.pallas import tpu as pltpu
from jax.experimental.pallas import tpu_sc as plsc

NUM_ENT = 1000000
NUM_REL = 1000
DIM = 64
B = 16384

NC = 2
NS = 16
NW = NC * NS
BW = B // NW
CH = 128
NCH = BW // CH

LN2 = 0.6931471805599453
SQRT2 = 1.4142135623730951


def _ln(x):
    xb = lax.bitcast_convert_type(x, jnp.int32)
    e = lax.shift_right_arithmetic(xb, 23) - 127
    mb = lax.bitwise_or(lax.bitwise_and(xb, 0x007FFFFF), 0x3F800000)
    m = lax.bitcast_convert_type(mb, jnp.float32)
    big = m > SQRT2
    m = jnp.where(big, m * 0.5, m)
    ef = e.astype(jnp.float32) + jnp.where(big, 1.0, 0.0)
    s = (m - 1.0) / (m + 1.0)
    u = s * s
    p = 2.0 + u * (0.6666666666 + u * (0.4000000897 + u * (0.2857142857 + u * 0.2222222222)))
    return ef * LN2 + s * p


def _energy_group(hmu, tmu, rmu, hsd, tsd, rsd):
    h_sig = jnp.abs(hsd) + 1e-6
    t_sig = jnp.abs(tsd) + 1e-6
    r_sig = jnp.abs(rsd) + 1e-6
    pred_sig = h_sig + r_sig + 1e-6
    inv_t = 1.0 / t_sig
    ratio = pred_sig * inv_t
    d = tmu - (hmu + rmu)
    return ratio + d * d * inv_t - _ln(ratio)


def _cat_body(a_ref, b_ref, o_ref, x_ref):
    x_ref[:DIM] = a_ref[...]
    x_ref[DIM:] = b_ref[...]
    o_ref[...] = x_ref[...].T


def _make_cat(n, cols):
    return pl.pallas_call(
        _cat_body,
        grid=(pl.cdiv(n, cols),),
        in_specs=[
            pl.BlockSpec((DIM, cols), lambda i: (0, i)),
            pl.BlockSpec((DIM, cols), lambda i: (0, i)),
        ],
        out_specs=pl.BlockSpec((cols, 2 * DIM), lambda i: (i, 0)),
        out_shape=jax.ShapeDtypeStruct((n, 2 * DIM), jnp.float32),
        scratch_shapes=[pltpu.VMEM((2 * DIM, cols), jnp.float32)],
    )


def _issue(ent, rel, hidx, tidx, ridx, c, bufs, sem):
    hb, tb, rb = bufs
    return [
        pltpu.async_copy(ent.at[hidx.at[c]], hb, sem),
        pltpu.async_copy(ent.at[tidx.at[c]], tb, sem),
        pltpu.async_copy(rel.at[ridx.at[c]], rb, sem),
    ]


def _sc_body(h_hbm, t_hbm, r_hbm, ent_hbm, rel_hbm, out_hbm,
             hidx, tidx, ridx, b0, b1, accs_f, out_v, sem0, sem1):
    wid = lax.axis_index("c") * NS + lax.axis_index("s")
    lanes = lax.iota(jnp.int32, 16)

    pltpu.sync_copy(h_hbm.at[wid], hidx)
    pltpu.sync_copy(t_hbm.at[wid], tidx)
    pltpu.sync_copy(r_hbm.at[wid], ridx)

    slots = ((b0, sem0), (b1, sem1))
    pending = _issue(ent_hbm, rel_hbm, hidx, tidx, ridx, 0, b0, sem0)

    for c in range(NCH):
        bufs, _ = slots[c % 2]
        hb, tb, rb = bufs
        for cp in pending:
            cp.wait()
        if c + 1 < NCH:
            nbufs, nsem = slots[(c + 1) % 2]
            pending = _issue(ent_hbm, rel_hbm, hidx, tidx, ridx, c + 1,
                             nbufs, nsem)

        def row_body(b, _):
            acc = jnp.zeros((16,), jnp.float32)
            for g in range(DIM // 16):
                mu = pl.ds(g * 16, 16)
                sd = pl.ds(DIM + g * 16, 16)
                acc = acc + _energy_group(
                    hb[b, mu], tb[b, mu], rb[b, mu],
                    hb[b, sd], tb[b, sd], rb[b, sd])
            accs_f[pl.ds(b * 16, 16)] = acc
            return 0

        lax.fori_loop(0, CH, row_body, 0)

        for bb in range(CH // 16):
            tot = jnp.zeros((16,), jnp.float32)
            for j in range(16):
                col = plsc.load_gather(accs_f, [bb * 256 + lanes * 16 + j])
                tot = tot + col
            out_v[pl.ds(c * CH + bb * 16, 16)] = 0.5 * tot

    pltpu.sync_copy(out_v, out_hbm.at[pl.ds(wid * BW, BW)])


@jax.jit
def _run(h2, t2, r2, ent_drift, ent_diff, rel_drift, rel_diff):
    ent_cat = _make_cat(NUM_ENT, 16384)(ent_drift.T, ent_diff.T)
    rel_cat = _make_cat(NUM_REL, NUM_REL)(rel_drift.T, rel_diff.T)

    mesh = plsc.VectorSubcoreMesh(core_axis_name="c", subcore_axis_name="s")
    buf = lambda: pltpu.VMEM((CH, 2 * DIM), jnp.float32)
    kfn = functools.partial(
        pl.kernel,
        out_type=jax.ShapeDtypeStruct((B,), jnp.float32),
        mesh=mesh,
        compiler_params=pltpu.CompilerParams(needs_layout_passes=False),
        scratch_types=[
            pltpu.VMEM((NCH, CH), jnp.int32),
            pltpu.VMEM((NCH, CH), jnp.int32),
            pltpu.VMEM((NCH, CH), jnp.int32),
            (buf(), buf(), buf()),
            (buf(), buf(), buf()),
            pltpu.VMEM((CH * 16,), jnp.float32),
            pltpu.VMEM((BW,), jnp.float32),
            pltpu.SemaphoreType.DMA,
            pltpu.SemaphoreType.DMA,
        ],
    )(_sc_body)
    return kfn(h2, t2, r2, ent_cat, rel_cat)


def kernel(h_idx, r_idx, t_idx, ent_drift, ent_diff, rel_drift, rel_diff):
    h2 = h_idx.astype(jnp.int32).reshape(NW, NCH, CH)
    t2 = t_idx.astype(jnp.int32).reshape(NW, NCH, CH)
    r2 = r_idx.astype(jnp.int32).reshape(NW, NCH, CH)
    return _run(h2, t2, r2, ent_drift, ent_diff, rel_drift, rel_diff)
